# baseline (speedup 1.0000x reference)
import jax
import jax.numpy as jnp
from jax.experimental import pallas as pl

OUT_SIZE = 7
SAMPLE = 2
TRANS_STD = 0.1
FINEST = 56.0
STRIDES = (4, 8, 16, 32)
C = 256
DF = 1024


def _levels(bboxes):
    scale = jnp.sqrt((bboxes[:, 2] - bboxes[:, 0]) * (bboxes[:, 3] - bboxes[:, 1]))
    lvl = jnp.floor(jnp.log2(scale / FINEST + 1e-6))
    return jnp.clip(lvl, 0, len(STRIDES) - 1).astype(jnp.int32)


def _pool(feats, Hs, Ws, scales, bboxes, lvls, offsets):
    PH, PW = jnp.meshgrid(jnp.arange(OUT_SIZE, dtype=jnp.float32),
                          jnp.arange(OUT_SIZE, dtype=jnp.float32), indexing='ij')
    iw = jnp.arange(SAMPLE, dtype=jnp.float32)
    ih = jnp.arange(SAMPLE, dtype=jnp.float32)

    def one(roi, lvl, off):
        x1, y1, x2, y2 = roi[0], roi[1], roi[2], roi[3]
        sc = scales[lvl]
        Wi = Ws[lvl]
        Hi = Hs[lvl]
        Wf = Wi.astype(jnp.float32)
        Hf = Hi.astype(jnp.float32)
        rsw = jnp.round(x1) * sc - 0.5
        rsh = jnp.round(y1) * sc - 0.5
        rw = jnp.maximum((jnp.round(x2) + 1.0) * sc - 0.5 - rsw, 0.1)
        rh = jnp.maximum((jnp.round(y2) + 1.0) * sc - 0.5 - rsh, 0.1)
        bw = rw / OUT_SIZE
        bh = rh / OUT_SIZE
        sbw = bw / SAMPLE
        sbh = bh / SAMPLE
        tx = off[0] * TRANS_STD
        ty = off[1] * TRANS_STD
        wst = PW * bw + rsw + tx * rw
        hst = PH * bh + rsh + ty * rh
        w = wst[:, :, None, None] + iw[None, None, None, :] * sbw
        h = hst[:, :, None, None] + ih[None, None, :, None] * sbh
        valid = (w >= -0.5) & (w <= Wf - 0.5) & (h >= -0.5) & (h <= Hf - 0.5)
        wc = jnp.clip(w, 0.0, Wf - 1.0)
        hc = jnp.clip(h, 0.0, Hf - 1.0)
        h0 = jnp.floor(hc).astype(jnp.int32)
        w0 = jnp.floor(wc).astype(jnp.int32)
        h1 = jnp.minimum(h0 + 1, Hi - 1)
        w1 = jnp.minimum(w0 + 1, Wi - 1)
        lh = (hc - h0.astype(jnp.float32))[..., None]
        lw = (wc - w0.astype(jnp.float32))[..., None]
        v00 = feats[lvl, :, h0, w0]
        v01 = feats[lvl, :, h0, w1]
        v10 = feats[lvl, :, h1, w0]
        v11 = feats[lvl, :, h1, w1]
        val = v00 * (1 - lh) * (1 - lw) + v01 * (1 - lh) * lw + v10 * lh * (1 - lw) + v11 * lh * lw
        val = val * valid[..., None].astype(val.dtype)
        cnt = jnp.sum(valid, axis=(2, 3)).astype(val.dtype)
        out = jnp.sum(val, axis=(2, 3)) / jnp.maximum(cnt, 1.0)[..., None]
        return jnp.transpose(out, (2, 0, 1))

    return jax.vmap(one)(bboxes, lvls, offsets)


def _copy_body(x_ref, o_ref):
    o_ref[...] = x_ref[...]


def kernel(feat0, feat1, feat2, feat3, bboxes, W1, b1, W2, b2, W3, b3):
    fl = (feat0, feat1, feat2, feat3)
    Hm, Wm = fl[0].shape[2], fl[0].shape[3]
    feats = jnp.zeros((4, C, Hm, Wm), feat0.dtype)
    for i, f in enumerate(fl):
        feats = feats.at[i, :, :f.shape[2], :f.shape[3]].set(f[0])
    Hs = jnp.array([f.shape[2] for f in fl], jnp.int32)
    Ws = jnp.array([f.shape[3] for f in fl], jnp.int32)
    scales = jnp.array([1.0 / s for s in STRIDES], jnp.float32)
    lvls = _levels(bboxes)
    N = bboxes.shape[0]
    zoff = jnp.zeros((N, 2, OUT_SIZE, OUT_SIZE), jnp.float32)
    x = _pool(feats, Hs, Ws, scales, bboxes, lvls, zoff)
    xf = x.reshape(N, -1)
    outs = []
    for i in range(4):
        hdd = jnp.maximum(xf @ W1[i].T + b1[i], 0.0)
        hdd = jnp.maximum(hdd @ W2[i].T + b2[i], 0.0)
        outs.append(hdd @ W3[i].T + b3[i])
    off = jnp.stack(outs, 0)[lvls, jnp.arange(N)].reshape(N, 2, OUT_SIZE, OUT_SIZE)
    y = _pool(feats, Hs, Ws, scales, bboxes, lvls, off)
    yf = y.reshape(N, C * OUT_SIZE * OUT_SIZE)
    yf = pl.pallas_call(
        _copy_body,
        grid=(N // 8,),
        in_specs=[pl.BlockSpec((8, C * OUT_SIZE * OUT_SIZE), lambda i: (i, 0))],
        out_specs=pl.BlockSpec((8, C * OUT_SIZE * OUT_SIZE), lambda i: (i, 0)),
        out_shape=jax.ShapeDtypeStruct(yf.shape, yf.dtype),
    )(yf)
    return yf.reshape(N, C, OUT_SIZE, OUT_SIZE)


# speedup vs baseline: 2.0467x; 2.0467x over previous
import functools

import jax
import jax.numpy as jnp
from jax import lax
from jax.experimental import pallas as pl
from jax.experimental.pallas import tpu as pltpu
from jax.experimental.pallas import tpu_sc as plsc

OUT_SIZE = 7
SAMPLE = 2
TRANS_STD = 0.1
FINEST = 56.0
STRIDES = (4, 8, 16, 32)
C = 256
DF = 1024
F = C * OUT_SIZE * OUT_SIZE
NOFF = 2 * OUT_SIZE * OUT_SIZE

NPAD = 1024
NW = 32
RPW = NPAD // NW
NCELL = OUT_SIZE * OUT_SIZE
TPC = SAMPLE * SAMPLE * 4
CHUNK_CELLS = 7
CHUNK = CHUNK_CELLS * TPC
NCHUNK = NCELL // CHUNK_CELLS
HMAX = 128
TROWS = 4 * HMAX * HMAX

_HS = (128, 64, 32, 16)
_WS = (128, 64, 32, 16)


def _levels(bboxes):
    scale = jnp.sqrt((bboxes[:, 2] - bboxes[:, 0]) * (bboxes[:, 3] - bboxes[:, 1]))
    lvl = jnp.floor(jnp.log2(scale / FINEST + 1e-6))
    return jnp.clip(lvl, 0, len(STRIDES) - 1).astype(jnp.int32)


def _taps(bboxes, lvls, off, nvalid):
    f32 = jnp.float32
    Np = bboxes.shape[0]
    PH, PW = jnp.meshgrid(jnp.arange(OUT_SIZE, dtype=f32),
                          jnp.arange(OUT_SIZE, dtype=f32), indexing='ij')
    iw = jnp.arange(SAMPLE, dtype=f32)
    ih = jnp.arange(SAMPLE, dtype=f32)
    scales = jnp.array([1.0 / s for s in STRIDES], f32)
    Hs = jnp.array(_HS, jnp.int32)
    Ws = jnp.array(_WS, jnp.int32)

    x1, y1, x2, y2 = bboxes[:, 0], bboxes[:, 1], bboxes[:, 2], bboxes[:, 3]
    sc = scales[lvls]
    Wi = Ws[lvls]
    Hi = Hs[lvls]
    Wf = Wi.astype(f32)
    Hf = Hi.astype(f32)
    rsw = jnp.round(x1) * sc - 0.5
    rsh = jnp.round(y1) * sc - 0.5
    rw = jnp.maximum((jnp.round(x2) + 1.0) * sc - 0.5 - rsw, 0.1)
    rh = jnp.maximum((jnp.round(y2) + 1.0) * sc - 0.5 - rsh, 0.1)
    bw = rw / OUT_SIZE
    bh = rh / OUT_SIZE
    sbw = bw / SAMPLE
    sbh = bh / SAMPLE
    tx = off[:, 0] * TRANS_STD
    ty = off[:, 1] * TRANS_STD
    e = lambda v: v[:, None, None]
    wst = PW[None] * e(bw) + e(rsw) + tx * e(rw)
    hst = PH[None] * e(bh) + e(rsh) + ty * e(rh)
    w = wst[..., None, None] + iw[None, None, None, None, :] * e(sbw)[..., None, None]
    h = hst[..., None, None] + ih[None, None, None, :, None] * e(sbh)[..., None, None]
    b = lambda v: jnp.broadcast_to(v, (Np, OUT_SIZE, OUT_SIZE, SAMPLE, SAMPLE))
    Wf5 = Wf[:, None, None, None, None]
    Hf5 = Hf[:, None, None, None, None]
    valid = (w >= -0.5) & (w <= Wf5 - 0.5) & (h >= -0.5) & (h <= Hf5 - 0.5)
    wc = jnp.clip(w, 0.0, Wf5 - 1.0)
    hc = jnp.clip(h, 0.0, Hf5 - 1.0)
    h0 = jnp.floor(hc).astype(jnp.int32)
    w0 = jnp.floor(wc).astype(jnp.int32)
    h1 = jnp.minimum(h0 + 1, Hi[:, None, None, None, None] - 1)
    w1 = jnp.minimum(w0 + 1, Wi[:, None, None, None, None] - 1)
    lh = hc - h0.astype(f32)
    lw = wc - w0.astype(f32)
    cnt = jnp.sum(valid, axis=(3, 4)).astype(f32)
    inv = (1.0 / jnp.maximum(cnt, 1.0))[..., None, None]
    vw = valid.astype(f32) * inv
    w00 = b((1 - lh) * (1 - lw)) * vw
    w01 = b((1 - lh) * lw) * vw
    w10 = b(lh * (1 - lw)) * vw
    w11 = b(lh * lw) * vw
    lvl5 = lvls[:, None, None, None, None]
    h0b, w0b, h1b, w1b = b(h0), b(w0), b(h1), b(w1)
    base = lvl5 * (HMAX * HMAX)
    i00 = base + h0b * HMAX + w0b
    i01 = base + h0b * HMAX + w1b
    i10 = base + h1b * HMAX + w0b
    i11 = base + h1b * HMAX + w1b
    gidx = jnp.stack([i00, i01, i10, i11], axis=-1)
    gw = jnp.stack([w00, w01, w10, w11], axis=-1)
    gw = gw * (jnp.arange(Np) < nvalid)[:, None, None, None, None, None].astype(f32)
    gidx = gidx.reshape(Np, NCHUNK, CHUNK)
    gw = gw.reshape(Np, NCHUNK, CHUNK)
    return gidx, gw


@functools.lru_cache(maxsize=1)
def _sc_pool_kernel():
    mesh = plsc.VectorSubcoreMesh(core_axis_name="c", subcore_axis_name="s")
    return functools.partial(
        pl.kernel,
        mesh=mesh,
        out_type=jax.ShapeDtypeStruct((NPAD, NCELL, C), jnp.float32),
        scratch_types=[
            pltpu.VMEM((NCHUNK, CHUNK), jnp.int32),
            pltpu.VMEM((NCHUNK, CHUNK), jnp.float32),
            pltpu.VMEM((CHUNK, C), jnp.float32),
            pltpu.VMEM((NCELL, C), jnp.float32),
            pltpu.SemaphoreType.DMA,
        ],
    )(_sc_pool_body)


def _sc_pool(table, gidx, gw):
    return _sc_pool_kernel()(table, gidx, gw)


def _sc_pool_body(table, gidx, gw, out, idx_v, w_v, rows_v, out_v, sem):
    wid = lax.axis_index("s") * 2 + lax.axis_index("c")
    rbase = wid * RPW

    def roi_body(i, carry):
        r = rbase + i
        pltpu.sync_copy(gidx.at[r], idx_v)
        pltpu.sync_copy(gw.at[r], w_v)

        def chunk_body(cc, carry2):
            pltpu.async_copy(table.at[idx_v.at[cc]], rows_v, sem).wait()

            def cell_body(j, carry3):
                wvec = w_v[cc, pl.ds(j * TPC, TPC)]
                ws = [wvec[t] for t in range(TPC)]
                for k in range(C // 16):
                    sl = pl.ds(k * 16, 16)
                    acc = rows_v[j * TPC, sl] * ws[0]
                    for t in range(1, TPC):
                        acc = acc + rows_v[j * TPC + t, sl] * ws[t]
                    out_v[cc * CHUNK_CELLS + j, sl] = acc
                return carry3

            lax.fori_loop(0, CHUNK_CELLS, cell_body, 0)
            return carry2

        lax.fori_loop(0, NCHUNK, chunk_body, 0)
        pltpu.sync_copy(out_v, out.at[r])
        return carry

    lax.fori_loop(0, RPW, roi_body, 0)


def _mm1_body(x_ref, w_ref, b_ref, o_ref):
    k = pl.program_id(2)
    nk = pl.num_programs(2)
    acc = lax.dot_general(x_ref[...], w_ref[...], (((1,), (1,)), ((), ())),
                          preferred_element_type=jnp.float32)

    @pl.when(k == 0)
    def _():
        o_ref[...] = acc

    @pl.when(k > 0)
    def _():
        o_ref[...] = o_ref[...] + acc

    @pl.when(k == nk - 1)
    def _():
        o_ref[...] = jnp.maximum(o_ref[...] + b_ref[...], 0.0)


def _mm_lvl_body(relu, x_ref, w_ref, b_ref, o_ref):
    acc = lax.dot_general(x_ref[...], w_ref[...], (((1,), (1,)), ((), ())),
                          preferred_element_type=jnp.float32)
    acc = acc + b_ref[...]
    if relu:
        acc = jnp.maximum(acc, 0.0)
    o_ref[...] = acc


def _mlp_full(xf, W1, b1, W2, b2, W3, b3):
    NB, KB = 256, 1792
    h1 = pl.pallas_call(
        _mm1_body,
        grid=(4, DF // NB, F // KB),
        in_specs=[
            pl.BlockSpec((NPAD, KB), lambda l, n, k: (0, k)),
            pl.BlockSpec((None, NB, KB), lambda l, n, k: (l, n, k)),
            pl.BlockSpec((None, 1, NB), lambda l, n, k: (l, 0, n)),
        ],
        out_specs=pl.BlockSpec((None, NPAD, NB), lambda l, n, k: (l, 0, n)),
        out_shape=jax.ShapeDtypeStruct((4, NPAD, DF), jnp.float32),
    )(xf, W1, b1[:, None, :])
    h2 = pl.pallas_call(
        functools.partial(_mm_lvl_body, True),
        grid=(4,),
        in_specs=[
            pl.BlockSpec((None, NPAD, DF), lambda l: (l, 0, 0)),
            pl.BlockSpec((None, DF, DF), lambda l: (l, 0, 0)),
            pl.BlockSpec((None, 1, DF), lambda l: (l, 0, 0)),
        ],
        out_specs=pl.BlockSpec((None, NPAD, DF), lambda l: (l, 0, 0)),
        out_shape=jax.ShapeDtypeStruct((4, NPAD, DF), jnp.float32),
    )(h1, W2, b2[:, None, :])
    W3p = jnp.zeros((4, 128, DF), jnp.float32).at[:, :NOFF].set(W3)
    b3p = jnp.zeros((4, 128), jnp.float32).at[:, :NOFF].set(b3)
    o3 = pl.pallas_call(
        functools.partial(_mm_lvl_body, False),
        grid=(4,),
        in_specs=[
            pl.BlockSpec((None, NPAD, DF), lambda l: (l, 0, 0)),
            pl.BlockSpec((None, 128, DF), lambda l: (l, 0, 0)),
            pl.BlockSpec((None, 1, 128), lambda l: (l, 0, 0)),
        ],
        out_specs=pl.BlockSpec((None, NPAD, 128), lambda l: (l, 0, 0)),
        out_shape=jax.ShapeDtypeStruct((4, NPAD, 128), jnp.float32),
    )(h2, W3p, b3p[:, None, :])
    return o3


def kernel(feat0, feat1, feat2, feat3, bboxes, W1, b1, W2, b2, W3, b3):
    N = bboxes.shape[0]
    fl = (feat0, feat1, feat2, feat3)
    table = jnp.zeros((4, HMAX, HMAX, C), jnp.float32)
    for i, f in enumerate(fl):
        table = table.at[i, :f.shape[2], :f.shape[3], :].set(
            jnp.transpose(f[0], (1, 2, 0)))
    table = table.reshape(TROWS, C)

    lvls = _levels(bboxes)
    bb = jnp.zeros((NPAD, 4), jnp.float32).at[:, 2:].set(8.0).at[:N].set(bboxes)
    lv = jnp.zeros((NPAD,), jnp.int32).at[:N].set(lvls)

    zoff = jnp.zeros((NPAD, 2, OUT_SIZE, OUT_SIZE), jnp.float32)
    gidx1, gw1 = _taps(bb, lv, zoff, N)
    pooled1 = _sc_pool(table, gidx1, gw1)

    xf = jnp.transpose(pooled1, (0, 2, 1)).reshape(NPAD, F)
    o3 = _mlp_full(xf, W1, b1, W2, b2, W3, b3)
    onehot = (lv[None, :] == jnp.arange(4)[:, None]).astype(jnp.float32)
    off = jnp.sum(o3[:, :, :NOFF] * onehot[:, :, None], axis=0)
    off = off.reshape(NPAD, 2, OUT_SIZE, OUT_SIZE)

    gidx2, gw2 = _taps(bb, lv, off, N)
    pooled2 = _sc_pool(table, gidx2, gw2)
    out = jnp.transpose(pooled2[:N], (0, 2, 1)).reshape(N, C, OUT_SIZE, OUT_SIZE)
    return out


# speedup vs baseline: 2.6229x; 1.2815x over previous
import functools

import jax
import jax.numpy as jnp
from jax import lax
from jax.experimental import pallas as pl
from jax.experimental.pallas import tpu as pltpu
from jax.experimental.pallas import tpu_sc as plsc

OUT_SIZE = 7
SAMPLE = 2
TRANS_STD = 0.1
FINEST = 56.0
STRIDES = (4, 8, 16, 32)
C = 256
DF = 1024
F = C * OUT_SIZE * OUT_SIZE
NOFF = 2 * OUT_SIZE * OUT_SIZE

NPAD = 1024
NW = 32
RPW = NPAD // NW
NCELL = OUT_SIZE * OUT_SIZE
TPC = SAMPLE * SAMPLE * 4
CHUNK_CELLS = 7
CHUNK = CHUNK_CELLS * TPC
NCHUNK = NCELL // CHUNK_CELLS
HMAX = 128
TROWS = 4 * HMAX * HMAX

_HS = (128, 64, 32, 16)
_WS = (128, 64, 32, 16)


def _levels(bboxes):
    scale = jnp.sqrt((bboxes[:, 2] - bboxes[:, 0]) * (bboxes[:, 3] - bboxes[:, 1]))
    lvl = jnp.floor(jnp.log2(scale / FINEST + 1e-6))
    return jnp.clip(lvl, 0, len(STRIDES) - 1).astype(jnp.int32)


def _taps(bboxes, lvls, off, nvalid):
    f32 = jnp.float32
    Np = bboxes.shape[0]
    PH, PW = jnp.meshgrid(jnp.arange(OUT_SIZE, dtype=f32),
                          jnp.arange(OUT_SIZE, dtype=f32), indexing='ij')
    iw = jnp.arange(SAMPLE, dtype=f32)
    ih = jnp.arange(SAMPLE, dtype=f32)
    scales = jnp.array([1.0 / s for s in STRIDES], f32)
    Hs = jnp.array(_HS, jnp.int32)
    Ws = jnp.array(_WS, jnp.int32)

    x1, y1, x2, y2 = bboxes[:, 0], bboxes[:, 1], bboxes[:, 2], bboxes[:, 3]
    sc = scales[lvls]
    Wi = Ws[lvls]
    Hi = Hs[lvls]
    Wf = Wi.astype(f32)
    Hf = Hi.astype(f32)
    rsw = jnp.round(x1) * sc - 0.5
    rsh = jnp.round(y1) * sc - 0.5
    rw = jnp.maximum((jnp.round(x2) + 1.0) * sc - 0.5 - rsw, 0.1)
    rh = jnp.maximum((jnp.round(y2) + 1.0) * sc - 0.5 - rsh, 0.1)
    bw = rw / OUT_SIZE
    bh = rh / OUT_SIZE
    sbw = bw / SAMPLE
    sbh = bh / SAMPLE
    tx = off[:, 0] * TRANS_STD
    ty = off[:, 1] * TRANS_STD
    e = lambda v: v[:, None, None]
    wst = PW[None] * e(bw) + e(rsw) + tx * e(rw)
    hst = PH[None] * e(bh) + e(rsh) + ty * e(rh)
    w = wst[..., None, None] + iw[None, None, None, None, :] * e(sbw)[..., None, None]
    h = hst[..., None, None] + ih[None, None, None, :, None] * e(sbh)[..., None, None]
    b = lambda v: jnp.broadcast_to(v, (Np, OUT_SIZE, OUT_SIZE, SAMPLE, SAMPLE))
    Wf5 = Wf[:, None, None, None, None]
    Hf5 = Hf[:, None, None, None, None]
    valid = (w >= -0.5) & (w <= Wf5 - 0.5) & (h >= -0.5) & (h <= Hf5 - 0.5)
    wc = jnp.clip(w, 0.0, Wf5 - 1.0)
    hc = jnp.clip(h, 0.0, Hf5 - 1.0)
    h0 = jnp.floor(hc).astype(jnp.int32)
    w0 = jnp.floor(wc).astype(jnp.int32)
    h1 = jnp.minimum(h0 + 1, Hi[:, None, None, None, None] - 1)
    w1 = jnp.minimum(w0 + 1, Wi[:, None, None, None, None] - 1)
    lh = hc - h0.astype(f32)
    lw = wc - w0.astype(f32)
    cnt = jnp.sum(valid, axis=(3, 4)).astype(f32)
    inv = (1.0 / jnp.maximum(cnt, 1.0))[..., None, None]
    vw = valid.astype(f32) * inv
    w00 = b((1 - lh) * (1 - lw)) * vw
    w01 = b((1 - lh) * lw) * vw
    w10 = b(lh * (1 - lw)) * vw
    w11 = b(lh * lw) * vw
    lvl5 = lvls[:, None, None, None, None]
    h0b, w0b, h1b, w1b = b(h0), b(w0), b(h1), b(w1)
    base = lvl5 * (HMAX * HMAX)
    i00 = base + h0b * HMAX + w0b
    i01 = base + h0b * HMAX + w1b
    i10 = base + h1b * HMAX + w0b
    i11 = base + h1b * HMAX + w1b
    gidx = jnp.stack([i00, i01, i10, i11], axis=-1)
    gw = jnp.stack([w00, w01, w10, w11], axis=-1)
    gw = gw * (jnp.arange(Np) < nvalid)[:, None, None, None, None, None].astype(f32)
    gidx = gidx.reshape(Np, NCHUNK, CHUNK)
    gw = gw.reshape(Np, NCHUNK, CHUNK)
    return gidx, gw


@functools.lru_cache(maxsize=1)
def _sc_pool_kernel():
    mesh = plsc.VectorSubcoreMesh(core_axis_name="c", subcore_axis_name="s")
    return functools.partial(
        pl.kernel,
        mesh=mesh,
        out_type=jax.ShapeDtypeStruct((NPAD, NCELL, C), jnp.float32),
        scratch_types=[
            pltpu.VMEM((NCHUNK, CHUNK), jnp.int32),
            pltpu.VMEM((NCHUNK, CHUNK), jnp.float32),
            pltpu.VMEM((CHUNK, C), jnp.float32),
            pltpu.VMEM((CHUNK, C), jnp.float32),
            pltpu.VMEM((NCELL, C), jnp.float32),
            pltpu.SemaphoreType.DMA,
            pltpu.SemaphoreType.DMA,
        ],
    )(_sc_pool_body)


def _sc_pool(table, gidx, gw):
    return _sc_pool_kernel()(table, gidx, gw)


def _sc_pool_body(table, gidx, gw, out, idx_v, w_v, rows_v0, rows_v1, out_v,
                  sem0, sem1):
    wid = lax.axis_index("s") * 2 + lax.axis_index("c")
    rbase = wid * RPW
    bufs = (rows_v0, rows_v1)
    sems = (sem0, sem1)

    def roi_body(i, carry):
        r = rbase + i
        pltpu.sync_copy(gidx.at[r], idx_v)
        pltpu.sync_copy(gw.at[r], w_v)

        copies = [pltpu.async_copy(table.at[idx_v.at[0]], bufs[0], sems[0])]
        for cc in range(NCHUNK):
            if cc + 1 < NCHUNK:
                copies.append(pltpu.async_copy(
                    table.at[idx_v.at[cc + 1]], bufs[(cc + 1) % 2],
                    sems[(cc + 1) % 2]))
            copies[cc].wait()
            rows = bufs[cc % 2]

            def cell_body(j, carry3, cc=cc, rows=rows):
                wvec = w_v[cc, pl.ds(j * TPC, TPC)]
                ws = [wvec[t] for t in range(TPC)]
                for k in range(C // 16):
                    sl = pl.ds(k * 16, 16)
                    acc = rows[j * TPC, sl] * ws[0]
                    for t in range(1, TPC):
                        acc = acc + rows[j * TPC + t, sl] * ws[t]
                    out_v[cc * CHUNK_CELLS + j, sl] = acc
                return carry3

            lax.fori_loop(0, CHUNK_CELLS, cell_body, 0)
        pltpu.sync_copy(out_v, out.at[r])
        return carry

    lax.fori_loop(0, RPW, roi_body, 0)


def _mm1_body(x_ref, w_ref, b_ref, o_ref):
    k = pl.program_id(2)
    nk = pl.num_programs(2)
    acc = lax.dot_general(x_ref[...], w_ref[...], (((1,), (1,)), ((), ())),
                          preferred_element_type=jnp.float32)

    @pl.when(k == 0)
    def _():
        o_ref[...] = acc

    @pl.when(k > 0)
    def _():
        o_ref[...] = o_ref[...] + acc

    @pl.when(k == nk - 1)
    def _():
        o_ref[...] = jnp.maximum(o_ref[...] + b_ref[...], 0.0)


def _mm_lvl_body(relu, x_ref, w_ref, b_ref, o_ref):
    acc = lax.dot_general(x_ref[...], w_ref[...], (((1,), (1,)), ((), ())),
                          preferred_element_type=jnp.float32)
    acc = acc + b_ref[...]
    if relu:
        acc = jnp.maximum(acc, 0.0)
    o_ref[...] = acc


def _mlp_full(xf, W1, b1, W2, b2, W3, b3):
    NB, KB = 256, 1792
    h1 = pl.pallas_call(
        _mm1_body,
        grid=(4, DF // NB, F // KB),
        in_specs=[
            pl.BlockSpec((NPAD, KB), lambda l, n, k: (0, k)),
            pl.BlockSpec((None, NB, KB), lambda l, n, k: (l, n, k)),
            pl.BlockSpec((None, 1, NB), lambda l, n, k: (l, 0, n)),
        ],
        out_specs=pl.BlockSpec((None, NPAD, NB), lambda l, n, k: (l, 0, n)),
        out_shape=jax.ShapeDtypeStruct((4, NPAD, DF), jnp.float32),
    )(xf, W1, b1[:, None, :])
    h2 = pl.pallas_call(
        functools.partial(_mm_lvl_body, True),
        grid=(4,),
        in_specs=[
            pl.BlockSpec((None, NPAD, DF), lambda l: (l, 0, 0)),
            pl.BlockSpec((None, DF, DF), lambda l: (l, 0, 0)),
            pl.BlockSpec((None, 1, DF), lambda l: (l, 0, 0)),
        ],
        out_specs=pl.BlockSpec((None, NPAD, DF), lambda l: (l, 0, 0)),
        out_shape=jax.ShapeDtypeStruct((4, NPAD, DF), jnp.float32),
    )(h1, W2, b2[:, None, :])
    W3p = jnp.zeros((4, 128, DF), jnp.float32).at[:, :NOFF].set(W3)
    b3p = jnp.zeros((4, 128), jnp.float32).at[:, :NOFF].set(b3)
    o3 = pl.pallas_call(
        functools.partial(_mm_lvl_body, False),
        grid=(4,),
        in_specs=[
            pl.BlockSpec((None, NPAD, DF), lambda l: (l, 0, 0)),
            pl.BlockSpec((None, 128, DF), lambda l: (l, 0, 0)),
            pl.BlockSpec((None, 1, 128), lambda l: (l, 0, 0)),
        ],
        out_specs=pl.BlockSpec((None, NPAD, 128), lambda l: (l, 0, 0)),
        out_shape=jax.ShapeDtypeStruct((4, NPAD, 128), jnp.float32),
    )(h2, W3p, b3p[:, None, :])
    return o3


def kernel(feat0, feat1, feat2, feat3, bboxes, W1, b1, W2, b2, W3, b3):
    N = bboxes.shape[0]
    fl = (feat0, feat1, feat2, feat3)
    table = jnp.zeros((4, HMAX, HMAX, C), jnp.float32)
    for i, f in enumerate(fl):
        table = table.at[i, :f.shape[2], :f.shape[3], :].set(
            jnp.transpose(f[0], (1, 2, 0)))
    table = table.reshape(TROWS, C)

    lvls = _levels(bboxes)
    bb = jnp.zeros((NPAD, 4), jnp.float32).at[:, 2:].set(8.0).at[:N].set(bboxes)
    lv = jnp.zeros((NPAD,), jnp.int32).at[:N].set(lvls)

    zoff = jnp.zeros((NPAD, 2, OUT_SIZE, OUT_SIZE), jnp.float32)
    gidx1, gw1 = _taps(bb, lv, zoff, N)
    pooled1 = _sc_pool(table, gidx1, gw1)

    xf = jnp.transpose(pooled1, (0, 2, 1)).reshape(NPAD, F)
    o3 = _mlp_full(xf, W1, b1, W2, b2, W3, b3)
    onehot = (lv[None, :] == jnp.arange(4)[:, None]).astype(jnp.float32)
    off = jnp.sum(o3[:, :, :NOFF] * onehot[:, :, None], axis=0)
    off = off.reshape(NPAD, 2, OUT_SIZE, OUT_SIZE)

    gidx2, gw2 = _taps(bb, lv, off, N)
    pooled2 = _sc_pool(table, gidx2, gw2)
    out = jnp.transpose(pooled2[:N], (0, 2, 1)).reshape(N, C, OUT_SIZE, OUT_SIZE)
    return out
